# baseline (speedup 1.0000x reference)
import functools

import jax
import jax.numpy as jnp
from jax import lax
from jax.experimental import pallas as pl
from jax.experimental.pallas import tpu as pltpu
from jax.experimental.pallas import tpu_sc as plsc

VOCAB = 100000
D = 1024
B = 4
L = 4096
BL = B * L

NC = 2
NS = 16
NLANES = 16
NW = NC * NS
LW = L // NW
C = 8
NLC = LW // C
VPR = D // NLANES
NV = C * VPR
GS = 3


def _body(ids_hbm, tok_hbm, pos_hbm, out_hbm, idx_v, tok_v, pos_v,
          gsem, ssem, psem):
    wid = lax.axis_index("s") * NC + lax.axis_index("c")
    l_base = wid * LW

    pltpu.sync_copy(ids_hbm.at[:, pl.ds(l_base, LW)], idx_v)

    def start_pos(q):
        s = q % GS
        return pltpu.async_copy(
            pos_hbm.at[pl.ds(l_base + q * C, C)], pos_v.at[s], psem.at[s])

    def start_gathers(q):
        s = q % GS
        descs = []
        for b in range(B):
            idx = idx_v.at[b, pl.ds(q * C, C)]
            descs.append(
                pltpu.async_copy(tok_hbm.at[idx], tok_v.at[s, b],
                                 gsem.at[s * B + b]))
        return descs

    def start_stores(q):
        s = q % GS
        descs = []
        for b in range(B):
            off = b * L + l_base + q * C
            descs.append(
                pltpu.async_copy(tok_v.at[s, b], out_hbm.at[pl.ds(off, C)],
                                 ssem.at[s * B + b]))
        return descs

    def fused_add(q):
        s = q % GS

        @plsc.parallel_loop(0, NV, unroll=8)
        def _add(i):
            r = lax.shift_right_logical(i, 6)
            col = pl.multiple_of(
                lax.shift_left(jnp.bitwise_and(i, VPR - 1), 4), NLANES)
            v = pos_v[s, r, pl.ds(col, NLANES)]
            for b in range(B):
                plsc.addupdate(tok_v.at[s, b, r, pl.ds(col, NLANES)], v)

    p_descs = [None] * NLC
    g_descs = [None] * NLC
    s_descs = [None] * NLC

    for q in range(2):
        p_descs[q] = start_pos(q)
        g_descs[q] = start_gathers(q)

    for q in range(NLC):
        if q + 2 < NLC:
            if q - 1 >= 0:
                for d in s_descs[q - 1]:
                    d.wait()
            p_descs[q + 2] = start_pos(q + 2)
            g_descs[q + 2] = start_gathers(q + 2)
        p_descs[q].wait()
        for d in g_descs[q]:
            d.wait()
        fused_add(q)
        s_descs[q] = start_stores(q)

    for q in range(NLC - 3, NLC):
        for d in s_descs[q]:
            d.wait()


@jax.jit
def _embed(input_ids, token_embed, pos_embed):
    mesh = plsc.VectorSubcoreMesh(
        core_axis_name="c", subcore_axis_name="s", num_cores=NC, num_subcores=NS
    )
    f = pl.kernel(
        _body,
        out_type=jax.ShapeDtypeStruct((BL, D), jnp.float32),
        mesh=mesh,
        scratch_types=[
            pltpu.VMEM((B, LW), jnp.int32),
            pltpu.VMEM((GS, B, C, D), jnp.float32),
            pltpu.VMEM((GS, C, D), jnp.float32),
            pltpu.SemaphoreType.DMA((GS * B,)),
            pltpu.SemaphoreType.DMA((GS * B,)),
            pltpu.SemaphoreType.DMA((GS,)),
        ],
    )
    return f(input_ids, token_embed, pos_embed)


def kernel(input_ids, token_embed, pos_embed):
    out = _embed(input_ids.astype(jnp.int32), token_embed, pos_embed)
    return out.reshape(B, L, D)


# speedup vs baseline: 1.0190x; 1.0190x over previous
import functools

import jax
import jax.numpy as jnp
from jax import lax
from jax.experimental import pallas as pl
from jax.experimental.pallas import tpu as pltpu
from jax.experimental.pallas import tpu_sc as plsc

VOCAB = 100000
D = 1024
B = 4
L = 4096
BL = B * L

NC = 2
NS = 16
NLANES = 16
NW = NC * NS
LW = L // NW
C = 8
NLC = LW // C
VPR = D // NLANES
NV = C * VPR
GS = 3


def _body(ids_hbm, tok_hbm, pos_hbm, out_hbm, idx_v, tok_v, pos_v,
          gsem, ssem, psem):
    wid = lax.axis_index("s") * NC + lax.axis_index("c")
    l_base = wid * LW

    pltpu.sync_copy(ids_hbm.at[:, pl.ds(l_base, LW)], idx_v)

    def start_pos(q):
        s = q % GS
        return pltpu.async_copy(
            pos_hbm.at[pl.ds(l_base + q * C, C)], pos_v.at[s], psem.at[s])

    def start_gathers(q):
        s = q % GS
        descs = []
        for b in range(B):
            idx = idx_v.at[b, pl.ds(q * C, C)]
            descs.append(
                pltpu.async_copy(tok_hbm.at[idx], tok_v.at[s, b],
                                 gsem.at[s * B + b], priority=1))
        return descs

    def start_stores(q):
        s = q % GS
        descs = []
        for b in range(B):
            off = b * L + l_base + q * C
            descs.append(
                pltpu.async_copy(tok_v.at[s, b], out_hbm.at[pl.ds(off, C)],
                                 ssem.at[s * B + b]))
        return descs

    def fused_add(q):
        s = q % GS

        @plsc.parallel_loop(0, NV, unroll=4)
        def _add(i):
            r = lax.shift_right_logical(i, 6)
            col = pl.multiple_of(
                lax.shift_left(jnp.bitwise_and(i, VPR - 1), 4), NLANES)
            v = pos_v[s, r, pl.ds(col, NLANES)]
            for b in range(B):
                plsc.addupdate(tok_v.at[s, b, r, pl.ds(col, NLANES)], v)

    p_descs = [None] * NLC
    g_descs = [None] * NLC
    s_descs = [None] * NLC

    for q in range(2):
        p_descs[q] = start_pos(q)
        g_descs[q] = start_gathers(q)

    for q in range(NLC):
        if q + 2 < NLC:
            if q - 1 >= 0:
                for d in s_descs[q - 1]:
                    d.wait()
            p_descs[q + 2] = start_pos(q + 2)
            g_descs[q + 2] = start_gathers(q + 2)
        p_descs[q].wait()
        for d in g_descs[q]:
            d.wait()
        fused_add(q)
        s_descs[q] = start_stores(q)

    for q in range(NLC - 3, NLC):
        for d in s_descs[q]:
            d.wait()


@jax.jit
def _embed(input_ids, token_embed, pos_embed):
    mesh = plsc.VectorSubcoreMesh(
        core_axis_name="c", subcore_axis_name="s", num_cores=NC, num_subcores=NS
    )
    f = pl.kernel(
        _body,
        out_type=jax.ShapeDtypeStruct((BL, D), jnp.float32),
        mesh=mesh,
        scratch_types=[
            pltpu.VMEM((B, LW), jnp.int32),
            pltpu.VMEM((GS, B, C, D), jnp.float32),
            pltpu.VMEM((GS, C, D), jnp.float32),
            pltpu.SemaphoreType.DMA((GS * B,)),
            pltpu.SemaphoreType.DMA((GS * B,)),
            pltpu.SemaphoreType.DMA((GS,)),
        ],
    )
    return f(input_ids, token_embed, pos_embed)


def kernel(input_ids, token_embed, pos_embed):
    out = _embed(input_ids.astype(jnp.int32), token_embed, pos_embed)
    return out.reshape(B, L, D)
